# baseline (speedup 1.0000x reference)
import jax
import jax.numpy as jnp
from jax.experimental import pallas as pl
from jax.experimental.pallas import tpu as pltpu

_D0, _D1, _K = 2048, 2048, 4
_Q, _C = 64, 128
_BI = 256
_NI = _D0 // _BI
_BQ = 8


def _blk(s):
    return jnp.where(s == 0, _NI - 1, s - 1)


def _issue_copy(x_hbm, scr, sem, slot, bi):
    row0 = bi * _BI
    for k in range(_NI):
        @pl.when(bi == k)
        def _():
            q0 = k * _BQ
            pltpu.make_async_copy(
                x_hbm.at[pl.ds(row0, _BI), pl.ds(q0, _Q - q0), :],
                scr.at[slot, :, pl.ds(q0, _Q - q0), :],
                sem.at[slot],
            ).start()


def _wait_copy(x_hbm, scr, sem, slot, bi):
    row0 = bi * _BI
    for k in range(_NI):
        @pl.when(bi == k)
        def _():
            q0 = k * _BQ
            pltpu.make_async_copy(
                x_hbm.at[pl.ds(row0, _BI), pl.ds(q0, _Q - q0), :],
                scr.at[slot, :, pl.ds(q0, _Q - q0), :],
                sem.at[slot],
            ).wait()


def _mask_kernel(x_hbm, o_ref, scr, sem):
    s = pl.program_id(0)
    bi = _blk(s)
    slot = jax.lax.rem(s, 2)

    @pl.when(s == 0)
    def _prologue():
        _issue_copy(x_hbm, scr, sem, 0, _blk(0))

    @pl.when(s + 1 < _NI)
    def _prefetch():
        _issue_copy(x_hbm, scr, sem, jax.lax.rem(s + 1, 2), _blk(s + 1))

    _wait_copy(x_hbm, scr, sem, slot, bi)

    rows = jax.lax.broadcasted_iota(jnp.int32, (_BI, _Q, _C), 0) + bi * _BI
    qs = jax.lax.broadcasted_iota(jnp.int32, (_BI, _Q, _C), 1)
    cs = jax.lax.broadcasted_iota(jnp.int32, (_BI, _Q, _C), 2)
    keep = (qs >> 2) * _C + cs >= rows
    o_ref[...] = jnp.where(keep, scr[slot], 0.0)


def kernel(w):
    x = (w.reshape(_D0, 16, _C, _K)
          .transpose(0, 1, 3, 2)
          .reshape(_D0, _Q, _C))
    out = pl.pallas_call(
        _mask_kernel,
        grid=(_NI,),
        in_specs=[pl.BlockSpec(memory_space=pltpu.MemorySpace.HBM)],
        out_specs=pl.BlockSpec((_BI, _Q, _C), lambda s: (_blk(s), 0, 0)),
        out_shape=jax.ShapeDtypeStruct((_D0, _Q, _C), jnp.float32),
        scratch_shapes=[
            pltpu.VMEM((2, _BI, _Q, _C), jnp.float32),
            pltpu.SemaphoreType.DMA((2,)),
        ],
    )(x)
    return (out.reshape(_D0, 16, _K, _C)
               .transpose(0, 1, 3, 2)
               .reshape(_D0, _D1, _K))


# speedup vs baseline: 1.0093x; 1.0093x over previous
import jax
import jax.numpy as jnp
from jax.experimental import pallas as pl
from jax.experimental.pallas import tpu as pltpu

_D0, _D1, _K = 2048, 2048, 4
_Q, _C = 64, 128
_BI = 256
_NI = _D0 // _BI
_BQ = 8


def _issue_copies(x_hbm, scr, sem, bi):
    slot = jax.lax.rem(bi, 2)
    row0 = bi * _BI
    for k in range(_NI):
        @pl.when(bi == k)
        def _():
            q0 = k * _BQ
            pltpu.make_async_copy(
                x_hbm.at[pl.ds(row0, _BI), pl.ds(q0, _Q - q0), :],
                scr.at[slot, :, pl.ds(q0, _Q - q0), :],
                sem.at[slot],
            ).start()


def _wait_copies(x_hbm, scr, sem, bi):
    slot = jax.lax.rem(bi, 2)
    row0 = bi * _BI
    for k in range(_NI):
        @pl.when(bi == k)
        def _():
            q0 = k * _BQ
            pltpu.make_async_copy(
                x_hbm.at[pl.ds(row0, _BI), pl.ds(q0, _Q - q0), :],
                scr.at[slot, :, pl.ds(q0, _Q - q0), :],
                sem.at[slot],
            ).wait()


def _mask_kernel(x_hbm, o_ref, scr, sem):
    bi = pl.program_id(0)

    @pl.when(bi == 0)
    def _prologue():
        _issue_copies(x_hbm, scr, sem, 0)

    @pl.when(bi + 1 < _NI)
    def _prefetch():
        _issue_copies(x_hbm, scr, sem, bi + 1)

    _wait_copies(x_hbm, scr, sem, bi)

    slot = jax.lax.rem(bi, 2)
    rows = jax.lax.broadcasted_iota(jnp.int32, (_BI, _Q, _C), 0) + bi * _BI
    qs = jax.lax.broadcasted_iota(jnp.int32, (_BI, _Q, _C), 1)
    cs = jax.lax.broadcasted_iota(jnp.int32, (_BI, _Q, _C), 2)
    keep = (qs >> 2) * _C + cs >= rows
    o_ref[...] = jnp.where(keep, scr[slot], 0.0)


def kernel(w):
    x = (w.reshape(_D0, 16, _C, _K)
          .transpose(0, 1, 3, 2)
          .reshape(_D0, _Q, _C))
    out = pl.pallas_call(
        _mask_kernel,
        grid=(_NI,),
        in_specs=[pl.BlockSpec(memory_space=pltpu.MemorySpace.HBM)],
        out_specs=pl.BlockSpec((_BI, _Q, _C), lambda bi: (bi, 0, 0)),
        out_shape=jax.ShapeDtypeStruct((_D0, _Q, _C), jnp.float32),
        scratch_shapes=[
            pltpu.VMEM((2, _BI, _Q, _C), jnp.float32),
            pltpu.SemaphoreType.DMA((2,)),
        ],
    )(x)
    return (out.reshape(_D0, 16, _K, _C)
               .transpose(0, 1, 3, 2)
               .reshape(_D0, _D1, _K))
